# baseline (speedup 1.0000x reference)
import jax
import jax.numpy as jnp
from jax import lax
from jax.experimental import pallas as pl
from jax.experimental.pallas import tpu as pltpu


def _stats_body(x_ref, xbf_ref, s1_ref, s2_ref, acc1_ref, acc2_ref):
    j = pl.program_id(1)

    @pl.when(j == 0)
    def _init():
        acc1_ref[...] = jnp.zeros_like(acc1_ref)
        acc2_ref[...] = jnp.zeros_like(acc2_ref)

    x = x_ref[...]
    acc1_ref[...] += jnp.sum(x, axis=0)
    acc2_ref[...] += jnp.sum(x * x, axis=0)
    xbf_ref[...] = x.astype(jnp.bfloat16)

    @pl.when(j == pl.num_programs(1) - 1)
    def _flush():
        s1_ref[...] = jnp.sum(acc1_ref[...], axis=1, keepdims=True)[None]
        s2_ref[...] = jnp.sum(acc2_ref[...], axis=1, keepdims=True)[None]


def _make_conv_body(img_blk, C, H, W, OC, inv_count, eps):
    HW = H * W

    def _body(x_ref, s1_ref, s2_ref, g_ref, b_ref, w_ref, o_ref, p_ref):
        s1 = s1_ref[0] + s1_ref[1]
        s2 = s2_ref[0] + s2_ref[1]
        mean = s1 * inv_count
        var = s2 * inv_count - mean * mean
        scale = g_ref[...] * lax.rsqrt(var + eps)
        shift = b_ref[...] - mean * scale

        lane = lax.broadcasted_iota(jnp.int32, (1, HW), 1)
        wpos = lane % W
        m_first = wpos != 0
        m_last = wpos != (W - 1)

        zf = jnp.float32(0)
        for b in range(img_blk):
            xb = x_ref[b].astype(jnp.float32)
            y = jnp.maximum(xb * scale + shift, 0.0).astype(jnp.bfloat16)
            p_ref[:C, :] = jnp.concatenate(
                [jnp.zeros((C, W), jnp.bfloat16), y[:, :HW - W]], axis=1)
            p_ref[C:2 * C, :] = y
            p_ref[2 * C:, :] = jnp.concatenate(
                [y[:, W:], jnp.zeros((C, W), jnp.bfloat16)], axis=1)
            zall = jnp.dot(w_ref[...], p_ref[...],
                           preferred_element_type=jnp.float32)
            z0 = zall[:OC]
            z1 = zall[OC:2 * OC]
            z2 = zall[2 * OC:]
            s0 = jnp.concatenate(
                [jnp.zeros((OC, 1), jnp.float32), z0[:, :HW - 1]], axis=1)
            s2_ = jnp.concatenate(
                [z2[:, 1:], jnp.zeros((OC, 1), jnp.float32)], axis=1)
            out = (z1 + jnp.where(m_first, s0, zf)
                   + jnp.where(m_last, s2_, zf))
            o_ref[b] = out.astype(jnp.bfloat16)

    return _body


def kernel(x_nchw, gamma, beta, weight_oihw, *, eps=1e-5):
    N, C, H, W = x_nchw.shape
    OC, Cin, KH, KW = weight_oihw.shape
    assert Cin == C and KH == 3 and KW == 3
    HW = H * W
    KC = 3 * C

    blk1 = 8 if N % 16 == 0 else 1
    half = N // (2 * blk1)
    blk2 = 2 if N % 4 == 0 else 1
    steps2 = N // blk2

    x_slab = x_nchw.reshape(N, C, HW).astype(jnp.float32)
    w_stack = (jnp.transpose(weight_oihw, (3, 0, 2, 1))
               .reshape(3 * OC, KC).astype(jnp.bfloat16))
    gamma2d = gamma.reshape(C, 1).astype(jnp.float32)
    beta2d = beta.reshape(C, 1).astype(jnp.float32)

    xbf, s1, s2 = pl.pallas_call(
        _stats_body,
        out_shape=(jax.ShapeDtypeStruct((N, C, HW), jnp.bfloat16),
                   jax.ShapeDtypeStruct((2, C, 1), jnp.float32),
                   jax.ShapeDtypeStruct((2, C, 1), jnp.float32)),
        grid=(2, half),
        in_specs=[pl.BlockSpec((blk1, C, HW),
                               lambda i, j: (i * half + j, 0, 0))],
        out_specs=(pl.BlockSpec((blk1, C, HW),
                                lambda i, j: (i * half + j, 0, 0)),
                   pl.BlockSpec((1, C, 1), lambda i, j: (i, 0, 0)),
                   pl.BlockSpec((1, C, 1), lambda i, j: (i, 0, 0))),
        scratch_shapes=[pltpu.VMEM((C, HW), jnp.float32),
                        pltpu.VMEM((C, HW), jnp.float32)],
        compiler_params=pltpu.CompilerParams(
            dimension_semantics=("parallel", "arbitrary")),
    )(x_slab)

    conv_body = _make_conv_body(blk2, C, H, W, OC,
                                1.0 / float(N * HW), eps)
    half2 = steps2 // 2
    out = pl.pallas_call(
        conv_body,
        out_shape=jax.ShapeDtypeStruct((N, OC, HW), jnp.bfloat16),
        grid=(2, half2),
        in_specs=[pl.BlockSpec((blk2, C, HW),
                               lambda i, j: (i * half2 + j, 0, 0)),
                  pl.BlockSpec((2, C, 1), lambda i, j: (0, 0, 0)),
                  pl.BlockSpec((2, C, 1), lambda i, j: (0, 0, 0)),
                  pl.BlockSpec((C, 1), lambda i, j: (0, 0)),
                  pl.BlockSpec((C, 1), lambda i, j: (0, 0)),
                  pl.BlockSpec((3 * OC, KC), lambda i, j: (0, 0))],
        out_specs=pl.BlockSpec((blk2, OC, HW),
                               lambda i, j: (i * half2 + j, 0, 0)),
        scratch_shapes=[pltpu.VMEM((KC, HW), jnp.bfloat16)],
        compiler_params=pltpu.CompilerParams(
            dimension_semantics=("parallel", "arbitrary")),
    )(xbf, s1, s2, gamma2d, beta2d, w_stack)

    return out.reshape(N, OC, H, W).astype(jnp.float32)


# speedup vs baseline: 1.0387x; 1.0387x over previous
import jax
import jax.numpy as jnp
from jax import lax
from jax.experimental import pallas as pl
from jax.experimental.pallas import tpu as pltpu


def _stats_body(x_ref, s1_ref, s2_ref, acc1_ref, acc2_ref):
    j = pl.program_id(1)

    @pl.when(j == 0)
    def _init():
        acc1_ref[...] = jnp.zeros_like(acc1_ref)
        acc2_ref[...] = jnp.zeros_like(acc2_ref)

    x = x_ref[...]
    acc1_ref[...] += jnp.sum(x, axis=0)
    acc2_ref[...] += jnp.sum(x * x, axis=0)

    @pl.when(j == pl.num_programs(1) - 1)
    def _flush():
        s1_ref[...] = jnp.sum(acc1_ref[...], axis=1, keepdims=True)[None]
        s2_ref[...] = jnp.sum(acc2_ref[...], axis=1, keepdims=True)[None]


def _make_conv_body(img_blk, C, H, W, OC, inv_count, eps):
    HW = H * W

    def _body(x_ref, s1_ref, s2_ref, g_ref, b_ref, w_ref, o_ref, p_ref):
        s1 = s1_ref[0] + s1_ref[1]
        s2 = s2_ref[0] + s2_ref[1]
        mean = s1 * inv_count
        var = s2 * inv_count - mean * mean
        scale = g_ref[...] * lax.rsqrt(var + eps)
        shift = b_ref[...] - mean * scale

        lane = lax.broadcasted_iota(jnp.int32, (1, HW), 1)
        wpos = lane % W
        m_first = wpos != 0
        m_last = wpos != (W - 1)

        zf = jnp.float32(0)
        for b in range(img_blk):
            y = jnp.maximum(x_ref[b] * scale + shift, 0.0).astype(jnp.bfloat16)
            p_ref[:C, :] = jnp.concatenate(
                [jnp.zeros((C, W), jnp.bfloat16), y[:, :HW - W]], axis=1)
            p_ref[C:2 * C, :] = y
            p_ref[2 * C:, :] = jnp.concatenate(
                [y[:, W:], jnp.zeros((C, W), jnp.bfloat16)], axis=1)
            zall = jnp.dot(w_ref[...], p_ref[...],
                           preferred_element_type=jnp.float32)
            z0 = zall[:OC]
            z1 = zall[OC:2 * OC]
            z2 = zall[2 * OC:]
            s0 = jnp.concatenate(
                [jnp.zeros((OC, 1), jnp.float32), z0[:, :HW - 1]], axis=1)
            s2_ = jnp.concatenate(
                [z2[:, 1:], jnp.zeros((OC, 1), jnp.float32)], axis=1)
            out = (z1 + jnp.where(m_first, s0, zf)
                   + jnp.where(m_last, s2_, zf))
            o_ref[b] = out.astype(jnp.bfloat16)

    return _body


def kernel(x_nchw, gamma, beta, weight_oihw, *, eps=1e-5):
    N, C, H, W = x_nchw.shape
    OC, Cin, KH, KW = weight_oihw.shape
    assert Cin == C and KH == 3 and KW == 3
    HW = H * W
    KC = 3 * C

    blk1 = 8 if N % 16 == 0 else 1
    half = N // (2 * blk1)
    blk2 = 2 if N % 4 == 0 else 1
    steps2 = N // blk2

    x_slab = x_nchw.reshape(N, C, HW).astype(jnp.float32)
    w_stack = (jnp.transpose(weight_oihw, (3, 0, 2, 1))
               .reshape(3 * OC, KC).astype(jnp.bfloat16))
    gamma2d = gamma.reshape(C, 1).astype(jnp.float32)
    beta2d = beta.reshape(C, 1).astype(jnp.float32)

    s1, s2 = pl.pallas_call(
        _stats_body,
        out_shape=(jax.ShapeDtypeStruct((2, C, 1), jnp.float32),
                   jax.ShapeDtypeStruct((2, C, 1), jnp.float32)),
        grid=(2, half),
        in_specs=[pl.BlockSpec((blk1, C, HW),
                               lambda i, j: (i * half + j, 0, 0))],
        out_specs=(pl.BlockSpec((1, C, 1), lambda i, j: (i, 0, 0)),
                   pl.BlockSpec((1, C, 1), lambda i, j: (i, 0, 0))),
        scratch_shapes=[pltpu.VMEM((C, HW), jnp.float32),
                        pltpu.VMEM((C, HW), jnp.float32)],
        compiler_params=pltpu.CompilerParams(
            dimension_semantics=("parallel", "arbitrary")),
    )(x_slab)

    conv_body = _make_conv_body(blk2, C, H, W, OC,
                                1.0 / float(N * HW), eps)
    half2 = steps2 // 2
    out = pl.pallas_call(
        conv_body,
        out_shape=jax.ShapeDtypeStruct((N, OC, HW), jnp.bfloat16),
        grid=(2, half2),
        in_specs=[pl.BlockSpec((blk2, C, HW),
                               lambda i, j: (i * half2 + j, 0, 0)),
                  pl.BlockSpec((2, C, 1), lambda i, j: (0, 0, 0)),
                  pl.BlockSpec((2, C, 1), lambda i, j: (0, 0, 0)),
                  pl.BlockSpec((C, 1), lambda i, j: (0, 0)),
                  pl.BlockSpec((C, 1), lambda i, j: (0, 0)),
                  pl.BlockSpec((3 * OC, KC), lambda i, j: (0, 0))],
        out_specs=pl.BlockSpec((blk2, OC, HW),
                               lambda i, j: (i * half2 + j, 0, 0)),
        scratch_shapes=[pltpu.VMEM((KC, HW), jnp.bfloat16)],
        compiler_params=pltpu.CompilerParams(
            dimension_semantics=("parallel", "parallel")),
    )(x_slab, s1, s2, gamma2d, beta2d, w_stack)

    return out.reshape(N, OC, H, W).astype(jnp.float32)
